# baseline (speedup 1.0000x reference)
import jax
import jax.numpy as jnp
from jax.experimental import pallas as pl
import jax.experimental.pallas.tpu as pltpu

B, S, D, E = 4, 2048, 4096, 64
S_CHUNK = 256
N_SCHUNKS = S // S_CHUNK
D_CHUNK = 256
N_DCHUNKS = D // D_CHUNK
PH1 = N_SCHUNKS
PH2 = N_SCHUNKS + N_DCHUNKS
NSTEPS = N_SCHUNKS + 2 * N_DCHUNKS


def _fused_kernel(state_ref, scale_ref, bias_ref, w1_ref, b1_ref,
                  w2_ref, b2_ref, emb_ref, cb_ref, probs_ref,
                  macc_ref, h_ref, g_ref, att_ref):
    i = pl.program_id(0)

    @pl.when(i == 0)
    def _init():
        macc_ref[...] = jnp.zeros_like(macc_ref)
        att_ref[...] = jnp.zeros_like(att_ref)

    @pl.when(i < PH1)
    def _mean_phase():
        macc_ref[...] += jnp.sum(state_ref[...], axis=1)

    @pl.when(i == PH1 - 1)
    def _layernorm():
        m = macc_ref[...] * (1.0 / S)
        mu = jnp.mean(m, axis=-1, keepdims=True)
        var = jnp.mean((m - mu) ** 2, axis=-1, keepdims=True)
        h = (m - mu) * jax.lax.rsqrt(var + 1e-5)
        h_ref[...] = h * scale_ref[...] + bias_ref[...]

    @pl.when((i >= PH1) & (i < PH2))
    def _mlp1_phase():
        z = jnp.dot(h_ref[...], w1_ref[...], preferred_element_type=jnp.float32)
        z = z + b1_ref[...]
        g = z * 0.5 * (1.0 + jax.lax.erf(z * 0.7071067811865476))
        g_ref[:, pl.ds((i - PH1) * D_CHUNK, D_CHUNK)] = g

    @pl.when(i >= PH2)
    def _mlp2_phase():
        p = jnp.dot(g_ref[...], w2_ref[...], preferred_element_type=jnp.float32)
        p = p + b2_ref[...]
        att_ref[...] += jnp.dot(p, emb_ref[...].T,
                                preferred_element_type=jnp.float32)

    @pl.when(i == NSTEPS - 1)
    def _finish():
        att = att_ref[...] + cb_ref[...]
        att = att - jnp.max(att, axis=-1, keepdims=True)
        ex = jnp.exp(att)
        probs_ref[...] = ex / jnp.sum(ex, axis=-1, keepdims=True)


def _clamp(lo, x, hi):
    return jnp.minimum(jnp.maximum(x, lo), hi)


def kernel(state, ln_scale, ln_bias, W1, b1, W2, b2, expert_emb, centrality, spatial, current_expert_idx):
    scale2 = ln_scale.reshape(1, D)
    bias2 = ln_bias.reshape(1, D)
    b1_2 = b1.reshape(1, D)
    b2_2 = b2.reshape(1, D)
    spatial_row = jax.lax.dynamic_index_in_dim(spatial, current_expert_idx, 0, keepdims=False)
    combined_bias = (centrality + spatial_row).reshape(1, E)

    probs = pl.pallas_call(
        _fused_kernel,
        grid=(NSTEPS,),
        in_specs=[
            pl.BlockSpec((B, S_CHUNK, D), lambda i: (0, jnp.minimum(i, N_SCHUNKS - 1), 0)),
            pl.BlockSpec((1, D), lambda i: (0, 0)),
            pl.BlockSpec((1, D), lambda i: (0, 0)),
            pl.BlockSpec((D, D_CHUNK), lambda i: (0, _clamp(0, i - PH1, N_DCHUNKS - 1))),
            pl.BlockSpec((1, D_CHUNK), lambda i: (0, _clamp(0, i - PH1, N_DCHUNKS - 1))),
            pl.BlockSpec((D, D_CHUNK), lambda i: (0, _clamp(0, i - PH2, N_DCHUNKS - 1))),
            pl.BlockSpec((1, D_CHUNK), lambda i: (0, _clamp(0, i - PH2, N_DCHUNKS - 1))),
            pl.BlockSpec((E, D_CHUNK), lambda i: (0, _clamp(0, i - PH2, N_DCHUNKS - 1))),
            pl.BlockSpec((1, E), lambda i: (0, 0)),
        ],
        out_specs=pl.BlockSpec((B, E), lambda i: (0, 0)),
        out_shape=jax.ShapeDtypeStruct((B, E), jnp.float32),
        scratch_shapes=[
            pltpu.VMEM((B, D), jnp.float32),
            pltpu.VMEM((B, D), jnp.float32),
            pltpu.VMEM((B, D), jnp.float32),
            pltpu.VMEM((B, E), jnp.float32),
        ],
    )(state, scale2, bias2, W1, b1_2, W2, b2_2, expert_emb, combined_bias)

    return probs


# speedup vs baseline: 1.1482x; 1.1482x over previous
import jax
import jax.numpy as jnp
from jax.experimental import pallas as pl
import jax.experimental.pallas.tpu as pltpu

B, S, D, E = 4, 2048, 4096, 64
S_CHUNK = 128
N_SCHUNKS = S // S_CHUNK
K_CHUNK = 512
N_KCHUNKS = D // K_CHUNK
PH1 = N_SCHUNKS
PH2 = N_SCHUNKS + N_KCHUNKS
NSTEPS = N_SCHUNKS + 2 * N_KCHUNKS


def _fused_kernel(state_ref, scale_ref, bias_ref, w1_ref, b1_ref,
                  w2_ref, b2_ref, emb_ref, cb_ref, probs_ref,
                  macc_ref, h_ref, zacc_ref, g_ref):
    i = pl.program_id(0)

    @pl.when(i == 0)
    def _init():
        macc_ref[...] = jnp.zeros_like(macc_ref)

    @pl.when(i < PH1)
    def _mean_phase():
        macc_ref[...] += jnp.sum(state_ref[...], axis=1)

    @pl.when(i == PH1 - 1)
    def _layernorm():
        m = macc_ref[...] * (1.0 / S)
        mu = jnp.mean(m, axis=-1, keepdims=True)
        var = jnp.mean((m - mu) ** 2, axis=-1, keepdims=True)
        h = (m - mu) * jax.lax.rsqrt(var + 1e-5)
        h_ref[...] = h * scale_ref[...] + bias_ref[...]
        zacc_ref[...] = jnp.zeros_like(zacc_ref)

    @pl.when((i >= PH1) & (i < PH2))
    def _mlp1_phase():
        hk = h_ref[:, pl.ds((i - PH1) * K_CHUNK, K_CHUNK)]
        zacc_ref[...] += jnp.dot(hk, w1_ref[...],
                                 preferred_element_type=jnp.float32)

    @pl.when(i == PH2 - 1)
    def _gelu():
        z = zacc_ref[...] + b1_ref[...]
        g_ref[...] = z * 0.5 * (1.0 + jax.lax.erf(z * 0.7071067811865476))
        zacc_ref[...] = jnp.zeros_like(zacc_ref)

    @pl.when(i >= PH2)
    def _mlp2_phase():
        gk = g_ref[:, pl.ds((i - PH2) * K_CHUNK, K_CHUNK)]
        zacc_ref[...] += jnp.dot(gk, w2_ref[...],
                                 preferred_element_type=jnp.float32)

    @pl.when(i == NSTEPS - 1)
    def _finish():
        p = zacc_ref[...] + b2_ref[...]
        att = jnp.dot(p, emb_ref[...].T, preferred_element_type=jnp.float32)
        att = att + cb_ref[...]
        att = att - jnp.max(att, axis=-1, keepdims=True)
        ex = jnp.exp(att)
        probs_ref[...] = ex / jnp.sum(ex, axis=-1, keepdims=True)


def _clamp(lo, x, hi):
    return jnp.minimum(jnp.maximum(x, lo), hi)


def kernel(state, ln_scale, ln_bias, W1, b1, W2, b2, expert_emb, centrality, spatial, current_expert_idx):
    scale2 = ln_scale.reshape(1, D)
    bias2 = ln_bias.reshape(1, D)
    b1_2 = b1.reshape(1, D)
    b2_2 = b2.reshape(1, D)
    spatial_row = jax.lax.dynamic_index_in_dim(spatial, current_expert_idx, 0, keepdims=False)
    combined_bias = (centrality + spatial_row).reshape(1, E)

    probs = pl.pallas_call(
        _fused_kernel,
        grid=(NSTEPS,),
        in_specs=[
            pl.BlockSpec((B, S_CHUNK, D), lambda i: (0, jnp.minimum(i, N_SCHUNKS - 1), 0)),
            pl.BlockSpec((1, D), lambda i: (0, 0)),
            pl.BlockSpec((1, D), lambda i: (0, 0)),
            pl.BlockSpec((K_CHUNK, D), lambda i: (_clamp(0, i - PH1, N_KCHUNKS - 1), 0)),
            pl.BlockSpec((1, D), lambda i: (0, 0)),
            pl.BlockSpec((K_CHUNK, D), lambda i: (_clamp(0, i - PH2, N_KCHUNKS - 1), 0)),
            pl.BlockSpec((1, D), lambda i: (0, 0)),
            pl.BlockSpec((E, D), lambda i: (0, 0)),
            pl.BlockSpec((1, E), lambda i: (0, 0)),
        ],
        out_specs=pl.BlockSpec((B, E), lambda i: (0, 0)),
        out_shape=jax.ShapeDtypeStruct((B, E), jnp.float32),
        scratch_shapes=[
            pltpu.VMEM((B, D), jnp.float32),
            pltpu.VMEM((B, D), jnp.float32),
            pltpu.VMEM((B, D), jnp.float32),
            pltpu.VMEM((B, D), jnp.float32),
        ],
    )(state, scale2, bias2, W1, b1_2, W2, b2_2, expert_emb, combined_bias)

    return probs
